# baseline (speedup 1.0000x reference)
import functools

import jax
import jax.numpy as jnp
from jax import lax
from jax.experimental import pallas as pl
from jax.experimental.pallas import tpu as pltpu
from jax.experimental.pallas import tpu_sc as plsc

_VOCAB = 1000000
_CLASSES = 1000
_D = 32
_B = 4096
_L = 200

_NC = 2
_NS = 16
_NW = _NC * _NS
_ROWS_PER_W = _B // _NW
_HALF = _L // 2
_NBUF = 4


_NTILE_FULL = 7812
_TPW = _NTILE_FULL // _NW
_NB1 = 4


def _sc_fmt_body(tt_hbm, tail_hbm, out_hbm, in_bufs, out_bufs, in_sems, out_sems):
    wid = lax.axis_index("s") * _NC + lax.axis_index("c")
    base_t = wid * _TPW
    iota = lax.iota(jnp.int32, 16)
    iota16 = iota + 16

    def fire_in(t, s):
        pltpu.async_copy(
            tt_hbm.at[:, pl.ds(t * 128, 128)], in_bufs.at[s], in_sems.at[s]
        )

    def wait_in(t, s):
        pltpu.make_async_copy(
            tt_hbm.at[:, pl.ds(t * 128, 128)], in_bufs.at[s], in_sems.at[s]
        ).wait()

    def fire_out(t, s):
        pltpu.async_copy(
            out_bufs.at[s], out_hbm.at[pl.ds(t * 4096, 4096)], out_sems.at[s]
        )

    def wait_out(t, s):
        pltpu.make_async_copy(
            out_bufs.at[s], out_hbm.at[pl.ds(t * 4096, 4096)], out_sems.at[s]
        ).wait()

    def transpose_tile(s, nj):
        def jbody(j, carry):
            jv = jnp.full((16,), j, jnp.int32)
            a = plsc.load_gather(in_bufs.at[s], [iota, jv])
            b = plsc.load_gather(in_bufs.at[s], [iota16, jv])
            out_bufs[s, pl.ds(j * 32, 16)] = a
            out_bufs[s, pl.ds(j * 32 + 16, 16)] = b
            return carry

        lax.fori_loop(0, nj, jbody, 0, unroll=4)

    for s in range(_NB1):
        fire_in(base_t + s, s)

    def outer(ti, carry):
        for s in range(_NB1):
            t = base_t + ti * _NB1 + s
            wait_in(t, s)

            @pl.when(ti > 0)
            def _():
                wait_out(t - _NB1, s)

            transpose_tile(s, 128)
            fire_out(t, s)

            @pl.when(ti < (_TPW // _NB1) - 1)
            def _():
                fire_in(t + _NB1, s)

        return carry

    lax.fori_loop(0, _TPW // _NB1, outer, 0)
    for s in range(_NB1):
        wait_out(base_t + _TPW - _NB1 + s, s)

    @pl.when(wid < 4)
    def _():
        t = _NTILE_FULL - 4 + wid
        fire_in(t, 0)
        wait_in(t, 0)
        transpose_tile(0, 128)
        fire_out(t, 0)
        wait_out(t, 0)

    @pl.when(wid == 31)
    def _():
        pltpu.sync_copy(tail_hbm, out_bufs.at[0, pl.ds(0, 64 * _D)])
        pltpu.sync_copy(
            out_bufs.at[0, pl.ds(0, 64 * _D)],
            out_hbm.at[pl.ds((_NTILE_FULL * 128) * _D, 64 * _D)],
        )


@functools.cache
def _sc_fmt():
    return pl.kernel(
        _sc_fmt_body,
        mesh=plsc.VectorSubcoreMesh(core_axis_name="c", subcore_axis_name="s"),
        compiler_params=pltpu.CompilerParams(
            use_tc_tiling_on_sc=True, needs_layout_passes=False
        ),
        out_type=jax.ShapeDtypeStruct((_VOCAB * _D,), jnp.float32),
        scratch_types=[
            pltpu.VMEM((_NB1, _D, 128), jnp.float32),
            pltpu.VMEM((_NB1, 128 * _D), jnp.float32),
            pltpu.SemaphoreType.DMA((_NB1,)),
            pltpu.SemaphoreType.DMA((_NB1,)),
        ],
    )


def _sc_pool_body(x_hbm, table_hbm, out_hbm, idx_v, bufs, acc, sems):
    wid = lax.axis_index("s") * _NC + lax.axis_index("c")
    row_base = wid * _ROWS_PER_W

    pltpu.sync_copy(x_hbm.at[pl.ds(row_base, _ROWS_PER_W)], idx_v)

    def fire(b, s):
        pltpu.async_copy(
            table_hbm.at[idx_v.at[b]],
            bufs.at[s],
            sems.at[s],
        )

    def drain(b, s):
        pltpu.make_async_copy(
            table_hbm.at[idx_v.at[b]],
            bufs.at[s],
            sems.at[s],
        ).wait()

    for s in range(_NBUF):
        fire(s, s)

    zeros = jnp.zeros((16,), jnp.float32)

    def outer(bb, carry):
        for s in range(_NBUF):
            b = bb * _NBUF + s
            drain(b, s)

            def body(r, c):
                a0, a1, a2, a3 = c
                a0 = a0 + bufs[s, 2 * r, pl.ds(0, 16)]
                a1 = a1 + bufs[s, 2 * r, pl.ds(16, 16)]
                a2 = a2 + bufs[s, 2 * r + 1, pl.ds(0, 16)]
                a3 = a3 + bufs[s, 2 * r + 1, pl.ds(16, 16)]
                return (a0, a1, a2, a3)

            nb = b + _NBUF

            @pl.when(nb < _ROWS_PER_W)
            def _():
                fire(nb, s)

            a0, a1, a2, a3 = lax.fori_loop(
                0, _L // 2, body, (zeros, zeros, zeros, zeros), unroll=2
            )
            acc[b, pl.ds(0, 16)] = a0 + a2
            acc[b, pl.ds(16, 16)] = a1 + a3
        return carry

    lax.fori_loop(0, _ROWS_PER_W // _NBUF, outer, 0)

    pltpu.sync_copy(acc, out_hbm.at[pl.ds(row_base, _ROWS_PER_W)])


@functools.cache
def _sc_pool():
    return pl.kernel(
        _sc_pool_body,
        mesh=plsc.VectorSubcoreMesh(core_axis_name="c", subcore_axis_name="s"),
        compiler_params=pltpu.CompilerParams(use_tc_tiling_on_sc=False),
        out_type=jax.ShapeDtypeStruct((_B, _D), jnp.float32),
        scratch_types=[
            pltpu.VMEM((_ROWS_PER_W, _L), jnp.int32),
            pltpu.VMEM((_NBUF, _L, _D), jnp.float32),
            pltpu.VMEM((_ROWS_PER_W, _D), jnp.float32),
            pltpu.SemaphoreType.DMA((_NBUF,)),
        ],
    )


def _tc_head_body(p_ref, w_ref, b_ref, o_ref):
    t = jnp.tanh(p_ref[...] * (1.0 / _L))
    o_ref[...] = (
        lax.dot_general(
            t, w_ref[...], (((1,), (1,)), ((), ())),
            preferred_element_type=jnp.float32,
        )
        + b_ref[...]
    )


def _tc_head(pooled, W, b2d):
    blk = 512
    return pl.pallas_call(
        _tc_head_body,
        grid=(_B // blk,),
        in_specs=[
            pl.BlockSpec((blk, _D), lambda i: (i, 0)),
            pl.BlockSpec((_CLASSES, _D), lambda i: (0, 0)),
            pl.BlockSpec((1, _CLASSES), lambda i: (0, 0)),
        ],
        out_specs=pl.BlockSpec((blk, _CLASSES), lambda i: (i, 0)),
        out_shape=jax.ShapeDtypeStruct((_B, _CLASSES), jnp.float32),
    )(pooled, W, b2d)


@jax.jit
def kernel(x, emb_table, W, b):
    tail = emb_table[_NTILE_FULL * 128 :, :].reshape(64 * _D)
    t_lin = _sc_fmt()(emb_table.T, tail)
    pooled = _sc_pool()(x, t_lin.reshape(_VOCAB, _D))
    return _tc_head(pooled, W, b.reshape(1, _CLASSES))


# speedup vs baseline: 1.0033x; 1.0033x over previous
import functools

import jax
import jax.numpy as jnp
from jax import lax
from jax.experimental import pallas as pl
from jax.experimental.pallas import tpu as pltpu
from jax.experimental.pallas import tpu_sc as plsc

_VOCAB = 1000000
_CLASSES = 1000
_D = 32
_B = 4096
_L = 200

_NC = 2
_NS = 16
_NW = _NC * _NS
_ROWS_PER_W = _B // _NW
_HALF = _L // 2
_NBUF = 4


_NTILE_FULL = 7812
_TPW = _NTILE_FULL // _NW
_NB1 = 4


def _sc_fmt_body(tt_hbm, tail_hbm, out_hbm, in_bufs, out_bufs, in_sems, out_sems):
    wid = lax.axis_index("s") * _NC + lax.axis_index("c")
    base_t = wid * _TPW
    iota = lax.iota(jnp.int32, 16)
    iota16 = iota + 16

    def fire_in(t, s):
        pltpu.async_copy(
            tt_hbm.at[:, pl.ds(t * 128, 128)],
            in_bufs.at[s, :, pl.ds(0, 128)],
            in_sems.at[s],
        )

    def wait_in(t, s):
        pltpu.make_async_copy(
            tt_hbm.at[:, pl.ds(t * 128, 128)],
            in_bufs.at[s, :, pl.ds(0, 128)],
            in_sems.at[s],
        ).wait()

    def fire_out(t, s):
        pltpu.async_copy(
            out_bufs.at[s], out_hbm.at[pl.ds(t * 4096, 4096)], out_sems.at[s]
        )

    def wait_out(t, s):
        pltpu.make_async_copy(
            out_bufs.at[s], out_hbm.at[pl.ds(t * 4096, 4096)], out_sems.at[s]
        ).wait()

    def transpose_tile(s, nj):
        def jbody(j, carry):
            jv = jnp.full((16,), j, jnp.int32)
            a = plsc.load_gather(in_bufs.at[s], [iota, jv])
            b = plsc.load_gather(in_bufs.at[s], [iota16, jv])
            out_bufs[s, pl.ds(j * 32, 16)] = a
            out_bufs[s, pl.ds(j * 32 + 16, 16)] = b
            return carry

        lax.fori_loop(0, nj, jbody, 0, unroll=4)

    for s in range(_NB1):
        fire_in(base_t + s, s)

    def outer(ti, carry):
        for s in range(_NB1):
            t = base_t + ti * _NB1 + s
            wait_in(t, s)

            @pl.when(ti > 0)
            def _():
                wait_out(t - _NB1, s)

            transpose_tile(s, 128)
            fire_out(t, s)

            @pl.when(ti < (_TPW // _NB1) - 1)
            def _():
                fire_in(t + _NB1, s)

        return carry

    lax.fori_loop(0, _TPW // _NB1, outer, 0)
    for s in range(_NB1):
        wait_out(base_t + _TPW - _NB1 + s, s)

    @pl.when(wid < 4)
    def _():
        t = _NTILE_FULL - 4 + wid
        fire_in(t, 0)
        wait_in(t, 0)
        transpose_tile(0, 128)
        fire_out(t, 0)
        wait_out(t, 0)

    @pl.when(wid == 31)
    def _():
        pltpu.sync_copy(tail_hbm, out_bufs.at[0, pl.ds(0, 64 * _D)])
        pltpu.sync_copy(
            out_bufs.at[0, pl.ds(0, 64 * _D)],
            out_hbm.at[pl.ds((_NTILE_FULL * 128) * _D, 64 * _D)],
        )


@functools.cache
def _sc_fmt():
    return pl.kernel(
        _sc_fmt_body,
        mesh=plsc.VectorSubcoreMesh(core_axis_name="c", subcore_axis_name="s"),
        compiler_params=pltpu.CompilerParams(
            use_tc_tiling_on_sc=True, needs_layout_passes=False
        ),
        out_type=jax.ShapeDtypeStruct((_VOCAB * _D,), jnp.float32),
        scratch_types=[
            pltpu.VMEM((_NB1, _D, 129), jnp.float32),
            pltpu.VMEM((_NB1, 128 * _D), jnp.float32),
            pltpu.SemaphoreType.DMA((_NB1,)),
            pltpu.SemaphoreType.DMA((_NB1,)),
        ],
    )


def _sc_pool_body(x_hbm, table_hbm, out_hbm, idx_v, bufs, acc, sems):
    wid = lax.axis_index("s") * _NC + lax.axis_index("c")
    row_base = wid * _ROWS_PER_W

    pltpu.sync_copy(x_hbm.at[pl.ds(row_base, _ROWS_PER_W)], idx_v)

    def fire(b, s):
        pltpu.async_copy(
            table_hbm.at[idx_v.at[b]],
            bufs.at[s],
            sems.at[s],
        )

    def drain(b, s):
        pltpu.make_async_copy(
            table_hbm.at[idx_v.at[b]],
            bufs.at[s],
            sems.at[s],
        ).wait()

    for s in range(_NBUF):
        fire(s, s)

    zeros = jnp.zeros((16,), jnp.float32)

    def outer(bb, carry):
        for s in range(_NBUF):
            b = bb * _NBUF + s
            drain(b, s)

            def body(r, c):
                a0, a1, a2, a3 = c
                a0 = a0 + bufs[s, 2 * r, pl.ds(0, 16)]
                a1 = a1 + bufs[s, 2 * r, pl.ds(16, 16)]
                a2 = a2 + bufs[s, 2 * r + 1, pl.ds(0, 16)]
                a3 = a3 + bufs[s, 2 * r + 1, pl.ds(16, 16)]
                return (a0, a1, a2, a3)

            nb = b + _NBUF

            @pl.when(nb < _ROWS_PER_W)
            def _():
                fire(nb, s)

            a0, a1, a2, a3 = lax.fori_loop(
                0, _L // 2, body, (zeros, zeros, zeros, zeros), unroll=2
            )
            acc[b, pl.ds(0, 16)] = a0 + a2
            acc[b, pl.ds(16, 16)] = a1 + a3
        return carry

    lax.fori_loop(0, _ROWS_PER_W // _NBUF, outer, 0)

    pltpu.sync_copy(acc, out_hbm.at[pl.ds(row_base, _ROWS_PER_W)])


@functools.cache
def _sc_pool():
    return pl.kernel(
        _sc_pool_body,
        mesh=plsc.VectorSubcoreMesh(core_axis_name="c", subcore_axis_name="s"),
        compiler_params=pltpu.CompilerParams(use_tc_tiling_on_sc=False),
        out_type=jax.ShapeDtypeStruct((_B, _D), jnp.float32),
        scratch_types=[
            pltpu.VMEM((_ROWS_PER_W, _L), jnp.int32),
            pltpu.VMEM((_NBUF, _L, _D), jnp.float32),
            pltpu.VMEM((_ROWS_PER_W, _D), jnp.float32),
            pltpu.SemaphoreType.DMA((_NBUF,)),
        ],
    )


def _tc_head_body(p_ref, w_ref, b_ref, o_ref):
    t = jnp.tanh(p_ref[...] * (1.0 / _L))
    o_ref[...] = (
        lax.dot_general(
            t, w_ref[...], (((1,), (1,)), ((), ())),
            preferred_element_type=jnp.float32,
        )
        + b_ref[...]
    )


def _tc_head(pooled, W, b2d):
    blk = 512
    return pl.pallas_call(
        _tc_head_body,
        grid=(_B // blk,),
        in_specs=[
            pl.BlockSpec((blk, _D), lambda i: (i, 0)),
            pl.BlockSpec((_CLASSES, _D), lambda i: (0, 0)),
            pl.BlockSpec((1, _CLASSES), lambda i: (0, 0)),
        ],
        out_specs=pl.BlockSpec((blk, _CLASSES), lambda i: (i, 0)),
        out_shape=jax.ShapeDtypeStruct((_B, _CLASSES), jnp.float32),
    )(pooled, W, b2d)


@jax.jit
def kernel(x, emb_table, W, b):
    tail = emb_table[_NTILE_FULL * 128 :, :].reshape(64 * _D)
    t_lin = _sc_fmt()(emb_table.T, tail)
    pooled = _sc_pool()(x, t_lin.reshape(_VOCAB, _D))
    return _tc_head(pooled, W, b.reshape(1, _CLASSES))


# speedup vs baseline: 1.1284x; 1.1247x over previous
import functools

import jax
import jax.numpy as jnp
from jax import lax
from jax.experimental import pallas as pl
from jax.experimental.pallas import tpu as pltpu
from jax.experimental.pallas import tpu_sc as plsc

_VOCAB = 1000000
_CLASSES = 1000
_D = 32
_B = 4096
_L = 200

_NC = 2
_NS = 16
_NW = _NC * _NS
_ROWS_PER_W = _B // _NW
_HALF = _L // 2
_NBUF = 4


_NTILE_FULL = 7812
_TPW = _NTILE_FULL // _NW
_NB1 = 4


def _sc_fmt_body(tt_hbm, tail_hbm, out_hbm, in_bufs, ob0, ob1, ob2, ob3, in_sems, out_sems):
    wid = lax.axis_index("s") * _NC + lax.axis_index("c")
    base_t = wid * _TPW
    outs = (ob0, ob1, ob2, ob3)

    def fire_in(t, s):
        pltpu.async_copy(
            tt_hbm.at[:, pl.ds(t * 128, 128)],
            in_bufs.at[s, :, pl.ds(0, 128)],
            in_sems.at[s],
        )

    def wait_in(t, s):
        pltpu.make_async_copy(
            tt_hbm.at[:, pl.ds(t * 128, 128)],
            in_bufs.at[s, :, pl.ds(0, 128)],
            in_sems.at[s],
        ).wait()

    def fire_out(t, s):
        pltpu.async_copy(
            outs[s], out_hbm.at[pl.ds(t * 4096, 4096)], out_sems.at[s]
        )

    def wait_out(t, s):
        pltpu.make_async_copy(
            outs[s], out_hbm.at[pl.ds(t * 4096, 4096)], out_sems.at[s]
        ).wait()

    iota32 = lax.iota(jnp.int32, 16) * 32

    def transpose_tile(s, nj):
        def jgbody(jg, carry):
            base = iota32 + jg * 512
            for d in range(_D):
                v = in_bufs[s, d, pl.ds(jg * 16, 16)]
                plsc.store_scatter(outs[s], [base + d], v)
            return carry

        lax.fori_loop(0, nj // 16, jgbody, 0, unroll=2)

    for s in range(_NB1):
        fire_in(base_t + s, s)

    def outer(ti, carry):
        for s in range(_NB1):
            t = base_t + ti * _NB1 + s
            wait_in(t, s)

            @pl.when(ti > 0)
            def _():
                wait_out(t - _NB1, s)

            transpose_tile(s, 128)
            fire_out(t, s)

            @pl.when(ti < (_TPW // _NB1) - 1)
            def _():
                fire_in(t + _NB1, s)

        return carry

    lax.fori_loop(0, _TPW // _NB1, outer, 0)
    for s in range(_NB1):
        wait_out(base_t + _TPW - _NB1 + s, s)

    @pl.when(wid < 4)
    def _():
        t = _NTILE_FULL - 4 + wid
        fire_in(t, 0)
        wait_in(t, 0)
        transpose_tile(0, 128)
        fire_out(t, 0)
        wait_out(t, 0)

    @pl.when(wid == 31)
    def _():
        pltpu.sync_copy(tail_hbm, ob0.at[pl.ds(0, 64 * _D)])
        pltpu.sync_copy(
            ob0.at[pl.ds(0, 64 * _D)],
            out_hbm.at[pl.ds((_NTILE_FULL * 128) * _D, 64 * _D)],
        )


@functools.cache
def _sc_fmt():
    return pl.kernel(
        _sc_fmt_body,
        mesh=plsc.VectorSubcoreMesh(core_axis_name="c", subcore_axis_name="s"),
        compiler_params=pltpu.CompilerParams(
            use_tc_tiling_on_sc=True, needs_layout_passes=False
        ),
        out_type=jax.ShapeDtypeStruct((_VOCAB * _D,), jnp.float32),
        scratch_types=[
            pltpu.VMEM((_NB1, _D, 129), jnp.float32),
            pltpu.VMEM((128 * _D,), jnp.float32),
            pltpu.VMEM((128 * _D,), jnp.float32),
            pltpu.VMEM((128 * _D,), jnp.float32),
            pltpu.VMEM((128 * _D,), jnp.float32),
            pltpu.SemaphoreType.DMA((_NB1,)),
            pltpu.SemaphoreType.DMA((_NB1,)),
        ],
    )


def _sc_pool_body(x_hbm, table_hbm, out_hbm, idx_v, bufs, acc, sems):
    wid = lax.axis_index("s") * _NC + lax.axis_index("c")
    row_base = wid * _ROWS_PER_W

    pltpu.sync_copy(x_hbm.at[pl.ds(row_base, _ROWS_PER_W)], idx_v)

    def fire(b, s):
        pltpu.async_copy(
            table_hbm.at[idx_v.at[b]],
            bufs.at[s],
            sems.at[s],
        )

    def drain(b, s):
        pltpu.make_async_copy(
            table_hbm.at[idx_v.at[b]],
            bufs.at[s],
            sems.at[s],
        ).wait()

    for s in range(_NBUF):
        fire(s, s)

    zeros = jnp.zeros((16,), jnp.float32)

    def outer(bb, carry):
        for s in range(_NBUF):
            b = bb * _NBUF + s
            drain(b, s)

            def body(r, c):
                a0, a1, a2, a3 = c
                a0 = a0 + bufs[s, 2 * r, pl.ds(0, 16)]
                a1 = a1 + bufs[s, 2 * r, pl.ds(16, 16)]
                a2 = a2 + bufs[s, 2 * r + 1, pl.ds(0, 16)]
                a3 = a3 + bufs[s, 2 * r + 1, pl.ds(16, 16)]
                return (a0, a1, a2, a3)

            nb = b + _NBUF

            @pl.when(nb < _ROWS_PER_W)
            def _():
                fire(nb, s)

            a0, a1, a2, a3 = lax.fori_loop(
                0, _L // 2, body, (zeros, zeros, zeros, zeros), unroll=2
            )
            acc[b, pl.ds(0, 16)] = a0 + a2
            acc[b, pl.ds(16, 16)] = a1 + a3
        return carry

    lax.fori_loop(0, _ROWS_PER_W // _NBUF, outer, 0)

    pltpu.sync_copy(acc, out_hbm.at[pl.ds(row_base, _ROWS_PER_W)])


@functools.cache
def _sc_pool():
    return pl.kernel(
        _sc_pool_body,
        mesh=plsc.VectorSubcoreMesh(core_axis_name="c", subcore_axis_name="s"),
        compiler_params=pltpu.CompilerParams(use_tc_tiling_on_sc=False),
        out_type=jax.ShapeDtypeStruct((_B, _D), jnp.float32),
        scratch_types=[
            pltpu.VMEM((_ROWS_PER_W, _L), jnp.int32),
            pltpu.VMEM((_NBUF, _L, _D), jnp.float32),
            pltpu.VMEM((_ROWS_PER_W, _D), jnp.float32),
            pltpu.SemaphoreType.DMA((_NBUF,)),
        ],
    )


_TBLK = 512


def _tc_fmt_body(tt_ref, eye_ref, o_ref):
    y = lax.dot_general(
        tt_ref[...], eye_ref[...], (((0,), (0,)), ((), ())),
        preferred_element_type=jnp.float32,
    )
    o_ref[...] = jnp.concatenate([y[j::4, :] for j in range(4)], axis=1)


def _tc_fmt(tt, eye):
    grid = (_VOCAB + 4 * _TBLK - 1) // (4 * _TBLK)
    return pl.pallas_call(
        _tc_fmt_body,
        grid=(grid,),
        in_specs=[
            pl.BlockSpec((_D, 4 * _TBLK), lambda i: (0, i)),
            pl.BlockSpec((_D, _D), lambda i: (0, 0)),
        ],
        out_specs=pl.BlockSpec((_TBLK, 128), lambda i: (i, 0)),
        out_shape=jax.ShapeDtypeStruct((grid * _TBLK, 128), jnp.float32),
    )(tt, eye)


def _tc_head_body(p_ref, w_ref, b_ref, o_ref):
    t = jnp.tanh(p_ref[...] * (1.0 / _L))
    o_ref[...] = (
        lax.dot_general(
            t, w_ref[...], (((1,), (1,)), ((), ())),
            preferred_element_type=jnp.float32,
        )
        + b_ref[...]
    )


def _tc_head(pooled, W, b2d):
    blk = 512
    return pl.pallas_call(
        _tc_head_body,
        grid=(_B // blk,),
        in_specs=[
            pl.BlockSpec((blk, _D), lambda i: (i, 0)),
            pl.BlockSpec((_CLASSES, _D), lambda i: (0, 0)),
            pl.BlockSpec((1, _CLASSES), lambda i: (0, 0)),
        ],
        out_specs=pl.BlockSpec((blk, _CLASSES), lambda i: (i, 0)),
        out_shape=jax.ShapeDtypeStruct((_B, _CLASSES), jnp.float32),
    )(pooled, W, b2d)


@jax.jit
def kernel(x, emb_table, W, b):
    tail = emb_table[_NTILE_FULL * 128 :, :].reshape(64 * _D)
    t_lin = _sc_fmt()(emb_table.T, tail)
    pooled = _sc_pool()(x, t_lin.reshape(_VOCAB, _D))
    return _tc_head(pooled, W, b.reshape(1, _CLASSES))
